# Initial kernel scaffold; baseline (speedup 1.0000x reference)
import jax
import jax.numpy as jnp
from jax.experimental import pallas as pl


def kernel(img, patch_indices):
    raise NotImplementedError("write your pallas kernel here")


# speedup vs baseline: 2.1664x; 2.1664x over previous
import functools

import jax
import jax.numpy as jnp
import numpy as np
from jax import lax
from jax.experimental import pallas as pl
from jax.experimental.pallas import tpu as pltpu
from jax.experimental.pallas import tpu_sc as plsc

H = 512
W = 512
PSZ = 16
CCH = 3
NB = 32
WW = W // PSZ
ROWS48 = CCH * PSZ
DPATCH = PSZ * PSZ * CCH

CH = 32
GSUB = 128

_i = np.arange(DPATCH)
_p1, _rem = _i // (PSZ * CCH), _i % (PSZ * CCH)
_p2, _c = _rem // CCH, _rem % CCH
_PERM = ((_c * PSZ + _p1) * PSZ + _p2).astype(np.int32)
_IPERM = np.empty(DPATCH, dtype=np.int32)
_IPERM[_PERM] = np.arange(DPATCH, dtype=np.int32)
_IPERM_CH = ((np.arange(CH * DPATCH) // DPATCH) * DPATCH
             + _IPERM[np.arange(CH * DPATCH) % DPATCH]).astype(np.int32)


@functools.lru_cache(maxsize=None)
def _make_sc_call(K: int):
    nfull, rem = divmod(K, CH)
    mesh = plsc.VectorSubcoreMesh(core_axis_name="c", subcore_axis_name="s")

    @functools.partial(
        pl.kernel,
        mesh=mesh,
        compiler_params=pltpu.CompilerParams(
            needs_layout_passes=False, use_tc_tiling_on_sc=False),
        out_type=jax.ShapeDtypeStruct((NB * K * DPATCH,), jnp.float32),
        scratch_types=[
            pltpu.VMEM((CH * ROWS48,), jnp.int32),
            pltpu.VMEM((CH * ROWS48, PSZ), jnp.float32),
            pltpu.VMEM((CH * DPATCH,), jnp.float32),
            pltpu.VMEM((CH * DPATCH,), jnp.int32),
            pltpu.SemaphoreType.DMA,
        ],
    )
    def sc_kernel(img_rows, ind, perm_tab, out, idx_v, in_v, out_v, perm_v,
                  sem):
        b = lax.axis_index("s") * 2 + lax.axis_index("c")
        pltpu.sync_copy(perm_tab, perm_v)

        def process(k0, L):
            nrows = L * ROWS48
            pltpu.sync_copy(ind.at[b, pl.ds(k0 * ROWS48, nrows)],
                            idx_v.at[pl.ds(0, nrows)])
            handles = []
            ngroups, gr = divmod(nrows, GSUB)
            for g in range(ngroups):
                handles.append(pltpu.async_copy(
                    img_rows.at[idx_v.at[pl.ds(g * GSUB, GSUB)]],
                    in_v.at[pl.ds(g * GSUB, GSUB)], sem))
            if gr:
                handles.append(pltpu.async_copy(
                    img_rows.at[idx_v.at[pl.ds(ngroups * GSUB, gr)]],
                    in_v.at[pl.ds(ngroups * GSUB, gr)], sem))
            for h in handles:
                h.wait()

            def pbody(v, carry):
                pv = perm_v[pl.ds(v * PSZ, PSZ)]
                plsc.store_scatter(out_v, [pv], in_v[v])
                return carry

            lax.fori_loop(0, nrows, pbody, 0, unroll=4)
            pltpu.sync_copy(
                out_v.at[pl.ds(0, L * DPATCH)],
                out.at[pl.ds((b * K + k0) * DPATCH, L * DPATCH)])

        def chunk_body(ci, carry):
            process(ci * CH, CH)
            return carry

        lax.fori_loop(0, nfull, chunk_body, 0)
        if rem:
            process(nfull * CH, rem)

    return sc_kernel


def _build_ind(patch_indices, K):
    r = (patch_indices // WW).astype(jnp.int32)
    q = (patch_indices % WW).astype(jnp.int32)
    b = jnp.arange(NB, dtype=jnp.int32)[:, None, None, None]
    c = jnp.arange(CCH, dtype=jnp.int32)[None, None, :, None]
    p1 = jnp.arange(PSZ, dtype=jnp.int32)[None, None, None, :]
    ind = ((b * CCH + c) * H + PSZ * r[None, :, None, None] + p1) * WW \
        + q[None, :, None, None]
    return ind.reshape(NB, K * ROWS48)


def kernel(img, patch_indices):
    K = patch_indices.shape[0]
    img_rows = img.reshape(NB * CCH * H * WW, PSZ)
    ind = _build_ind(patch_indices, K)
    out_flat = _make_sc_call(K)(img_rows, ind, jnp.asarray(_IPERM_CH))
    return out_flat.reshape(NB, K, DPATCH)


# speedup vs baseline: 2.4934x; 1.1510x over previous
import functools

import jax
import jax.numpy as jnp
import numpy as np
from jax import lax
from jax.experimental import pallas as pl
from jax.experimental.pallas import tpu as pltpu
from jax.experimental.pallas import tpu_sc as plsc

H = 512
W = 512
PSZ = 16
CCH = 3
NB = 32
WW = W // PSZ
ROWS48 = CCH * PSZ
DPATCH = PSZ * PSZ * CCH

CH = 32
GSUB = 128

_i = np.arange(DPATCH)
_p1, _rem = _i // (PSZ * CCH), _i % (PSZ * CCH)
_p2, _c = _rem // CCH, _rem % CCH
_PERM = ((_c * PSZ + _p1) * PSZ + _p2).astype(np.int32)
_IPERM = np.empty(DPATCH, dtype=np.int32)
_IPERM[_PERM] = np.arange(DPATCH, dtype=np.int32)
_IPERM_CH = ((np.arange(CH * DPATCH) // DPATCH) * DPATCH
             + _IPERM[np.arange(CH * DPATCH) % DPATCH]).astype(np.int32)


@functools.lru_cache(maxsize=None)
def _make_sc_call(K: int):
    nfull, rem = divmod(K, CH)
    chunks = [(i * CH, CH) for i in range(nfull)]
    if rem:
        chunks.append((nfull * CH, rem))
    n = len(chunks)
    mesh = plsc.VectorSubcoreMesh(core_axis_name="c", subcore_axis_name="s")

    @functools.partial(
        pl.kernel,
        mesh=mesh,
        compiler_params=pltpu.CompilerParams(
            needs_layout_passes=False, use_tc_tiling_on_sc=False),
        out_type=jax.ShapeDtypeStruct((NB * K * DPATCH,), jnp.float32),
        scratch_types=[
            [pltpu.VMEM((CH * ROWS48,), jnp.int32) for _ in range(2)],
            [pltpu.VMEM((CH * ROWS48, PSZ), jnp.float32) for _ in range(2)],
            [pltpu.VMEM((CH * DPATCH,), jnp.float32) for _ in range(2)],
            pltpu.VMEM((CH * DPATCH,), jnp.int32),
            [pltpu.SemaphoreType.DMA for _ in range(6)],
        ],
    )
    def sc_kernel(img_rows, ind, perm_tab, out, idx_v, in_v, out_v, perm_v,
                  sems):
        b = lax.axis_index("s") * 2 + lax.axis_index("c")
        sem_idx, sem_g, sem_wb = sems[0:2], sems[2:4], sems[4:6]
        pltpu.sync_copy(perm_tab, perm_v)

        def fire_idx(i):
            k0, L = chunks[i]
            p = i % 2
            return pltpu.async_copy(
                ind.at[pl.ds((b * K + k0) * ROWS48, L * ROWS48)],
                idx_v[p].at[pl.ds(0, L * ROWS48)], sem_idx[p])

        def fire_gathers(i):
            _, L = chunks[i]
            p = i % 2
            handles = []
            ngroups, gr = divmod(L * ROWS48, GSUB)
            for g in range(ngroups):
                handles.append(pltpu.async_copy(
                    img_rows.at[idx_v[p].at[pl.ds(g * GSUB, GSUB)]],
                    in_v[p].at[pl.ds(g * GSUB, GSUB)], sem_g[p]))
            if gr:
                handles.append(pltpu.async_copy(
                    img_rows.at[idx_v[p].at[pl.ds(ngroups * GSUB, gr)]],
                    in_v[p].at[pl.ds(ngroups * GSUB, gr)], sem_g[p]))
            return handles

        def permute(i):
            _, L = chunks[i]
            p = i % 2

            def pbody(v, carry):
                pv = perm_v[pl.ds(v * PSZ, PSZ)]
                plsc.store_scatter(out_v[p], [pv], in_v[p][v])
                return carry

            lax.fori_loop(0, L * ROWS48, pbody, 0, unroll=4)

        def fire_wb(i):
            k0, L = chunks[i]
            p = i % 2
            return pltpu.async_copy(
                out_v[p].at[pl.ds(0, L * DPATCH)],
                out.at[pl.ds((b * K + k0) * DPATCH, L * DPATCH)], sem_wb[p])

        h_idx = [None] * n
        h_g = [None] * n
        h_wb = [None] * n
        h_idx[0] = fire_idx(0)
        h_idx[0].wait()
        h_g[0] = fire_gathers(0)
        if n > 1:
            h_idx[1] = fire_idx(1)
        for i in range(n):
            if i + 1 < n:
                h_idx[i + 1].wait()
                h_g[i + 1] = fire_gathers(i + 1)
            for h in h_g[i]:
                h.wait()
            if i + 2 < n:
                h_idx[i + 2] = fire_idx(i + 2)
            if i >= 2:
                h_wb[i - 2].wait()
            permute(i)
            h_wb[i] = fire_wb(i)
        for i in range(max(0, n - 2), n):
            h_wb[i].wait()

    return sc_kernel


def _build_ind(patch_indices, K):
    r = (patch_indices // WW).astype(jnp.int32)
    q = (patch_indices % WW).astype(jnp.int32)
    b = jnp.arange(NB, dtype=jnp.int32)[:, None, None, None]
    c = jnp.arange(CCH, dtype=jnp.int32)[None, None, :, None]
    p1 = jnp.arange(PSZ, dtype=jnp.int32)[None, None, None, :]
    ind = ((b * CCH + c) * H + PSZ * r[None, :, None, None] + p1) * WW \
        + q[None, :, None, None]
    return ind.reshape(NB * K * ROWS48)


def kernel(img, patch_indices):
    K = patch_indices.shape[0]
    img_rows = img.reshape(NB * CCH * H * WW, PSZ)
    ind = _build_ind(patch_indices, K)
    out_flat = _make_sc_call(K)(img_rows, ind, jnp.asarray(_IPERM_CH))
    return out_flat.reshape(NB, K, DPATCH)


# speedup vs baseline: 3.1357x; 1.2576x over previous
import functools

import jax
import jax.numpy as jnp
import numpy as np
from jax import lax
from jax.experimental import pallas as pl
from jax.experimental.pallas import tpu as pltpu
from jax.experimental.pallas import tpu_sc as plsc

H = 512
W = 512
PSZ = 16
CCH = 3
NB = 32
WW = W // PSZ
NR = H // PSZ
ROWS48 = CCH * PSZ
DPATCH = PSZ * PSZ * CCH

CH = 32
RING = 3
CHB = CH * DPATCH

_i = np.arange(DPATCH)
_p1, _rem = _i // (PSZ * CCH), _i % (PSZ * CCH)
_p2, _c = _rem // CCH, _rem % CCH
_PERM = ((_c * PSZ + _p1) * PSZ + _p2).astype(np.int32)
_IPERM = np.empty(DPATCH, dtype=np.int32)
_IPERM[_PERM] = np.arange(DPATCH, dtype=np.int32)


def _vextract(ref, i):
    base = (i >> 4) << 4
    v = ref[pl.ds(base, PSZ)]
    lane = i - base
    sel = jnp.where(lax.iota(jnp.int32, PSZ) == lane, v, 0)
    return jnp.sum(sel)


@functools.lru_cache(maxsize=None)
def _make_sc_call(K: int, Kp: int):
    nblk, rem = divmod(K, CH)
    mesh = plsc.VectorSubcoreMesh(core_axis_name="c", subcore_axis_name="s")

    @functools.partial(
        pl.kernel,
        mesh=mesh,
        compiler_params=pltpu.CompilerParams(
            needs_layout_passes=False, use_tc_tiling_on_sc=True),
        out_type=jax.ShapeDtypeStruct((NB * K * DPATCH,), jnp.float32),
        scratch_types=[
            pltpu.VMEM((ROWS48, W), jnp.float32),
            pltpu.VMEM((ROWS48, W), jnp.float32),
            pltpu.VMEM((RING * CHB,), jnp.float32),
            pltpu.VMEM((Kp,), jnp.int32),
            pltpu.VMEM((ROWS48,), jnp.int32),
            pltpu.VMEM((DPATCH,), jnp.int32),
            [pltpu.SemaphoreType.DMA for _ in range(4)],
        ],
    )
    def sc_kernel(img2d, qcol, starts, iperm, out, strip_a, strip_b, ring,
                  qcol_v, starts_v, iperm_v, sems):
        b = lax.axis_index("s") * 2 + lax.axis_index("c")
        sem_sa, sem_sb, sem_wb, sem_rem = sems
        pltpu.sync_copy(qcol, qcol_v)
        pltpu.sync_copy(starts, starts_v)
        pltpu.sync_copy(iperm, iperm_v)

        def fire_strip(r, strip, sem):
            return [
                pltpu.async_copy(
                    img2d.at[pl.ds((b * CCH + c) * H + PSZ * r, PSZ), :],
                    strip.at[pl.ds(c * PSZ, PSZ), :], sem)
                for c in range(CCH)
            ]

        def drain_strip(strip, sem):
            for c in range(CCH):
                pltpu.make_async_copy(
                    img2d.at[pl.ds(0, PSZ), :],
                    strip.at[pl.ds(c * PSZ, PSZ), :], sem).wait()

        def process(r, strip):
            s0 = _vextract(starts_v, r)
            s1 = _vextract(starts_v, r + 1)

            def pbody(k, carry):
                cb = _vextract(qcol_v, k)
                km = k - (k // (RING * CH)) * (RING * CH)
                base = km * DPATCH
                for j in range(ROWS48):
                    ipj = iperm_v[pl.ds(j * PSZ, PSZ)] + base
                    vec = strip[j, pl.ds(cb, PSZ)]
                    plsc.store_scatter(ring, [ipj], vec)
                return carry

            lax.fori_loop(s0, s1, pbody, 0)

            def fbody(blk, carry):
                @pl.when(blk >= RING - 2)
                def _():
                    pltpu.make_async_copy(
                        out.at[pl.ds(0, CHB)], ring.at[pl.ds(0, CHB)],
                        sem_wb).wait()
                slot = blk - (blk // RING) * RING
                pltpu.async_copy(
                    ring.at[pl.ds(slot * CHB, CHB)],
                    out.at[pl.ds((b * K + blk * CH) * DPATCH, CHB)], sem_wb)
                return carry

            lax.fori_loop(s0 // CH, s1 // CH, fbody, 0)

        h0 = fire_strip(0, strip_a, sem_sa)
        del h0

        def srbody(rr, carry):
            r0 = rr * 2
            hb = fire_strip(r0 + 1, strip_b, sem_sb)
            drain_strip(strip_a, sem_sa)
            process(r0, strip_a)

            @pl.when(r0 + 2 < NR)
            def _():
                fire_strip(r0 + 2, strip_a, sem_sa)

            for h in hb:
                h.wait()
            process(r0 + 1, strip_b)
            return carry

        lax.fori_loop(0, NR // 2, srbody, 0)

        if rem:
            slot = nblk - (nblk // RING) * RING
            pltpu.async_copy(
                ring.at[pl.ds(slot * CHB, rem * DPATCH)],
                out.at[pl.ds((b * K + nblk * CH) * DPATCH, rem * DPATCH)],
                sem_rem)
        for _ in range(nblk - max(0, nblk - (RING - 2))):
            pltpu.make_async_copy(
                out.at[pl.ds(0, CHB)], ring.at[pl.ds(0, CHB)], sem_wb).wait()
        if rem:
            pltpu.make_async_copy(
                out.at[pl.ds(0, rem * DPATCH)],
                ring.at[pl.ds(0, rem * DPATCH)], sem_rem).wait()

    return sc_kernel


def kernel(img, patch_indices):
    K = patch_indices.shape[0]
    Kp = ((K + PSZ) // PSZ) * PSZ
    img2d = img.reshape(NB * CCH * H, W)
    q = (patch_indices % WW).astype(jnp.int32)
    qcol = jnp.zeros((Kp,), jnp.int32).at[:K].set(q * PSZ)
    r = (patch_indices // WW).astype(jnp.int32)
    starts = jnp.searchsorted(r, jnp.arange(ROWS48, dtype=jnp.int32),
                              side="left").astype(jnp.int32)
    out_flat = _make_sc_call(K, Kp)(img2d, qcol, starts,
                                    jnp.asarray(_IPERM))
    return out_flat.reshape(NB, K, DPATCH)


# speedup vs baseline: 3.5219x; 1.1232x over previous
import functools

import jax
import jax.numpy as jnp
import numpy as np
from jax import lax
from jax.experimental import pallas as pl
from jax.experimental.pallas import tpu as pltpu
from jax.experimental.pallas import tpu_sc as plsc

H = 512
W = 512
PSZ = 16
CCH = 3
NB = 32
WW = W // PSZ
NR = H // PSZ
ROWS48 = CCH * PSZ
DPATCH = PSZ * PSZ * CCH

CH = 32
RING = 3
CHB = CH * DPATCH

_i = np.arange(DPATCH)
_p1, _rem = _i // (PSZ * CCH), _i % (PSZ * CCH)
_p2, _c = _rem // CCH, _rem % CCH
_PERM = ((_c * PSZ + _p1) * PSZ + _p2).astype(np.int32)
_IPERM = np.empty(DPATCH, dtype=np.int32)
_IPERM[_PERM] = np.arange(DPATCH, dtype=np.int32)


def _vextract(ref, i):
    base = (i >> 4) << 4
    v = ref[pl.ds(base, PSZ)]
    lane = i - base
    sel = jnp.where(lax.iota(jnp.int32, PSZ) == lane, v, 0)
    return jnp.sum(sel)


@functools.lru_cache(maxsize=None)
def _make_sc_call(K: int, Kp: int):
    nblk, rem = divmod(K, CH)
    mesh = plsc.VectorSubcoreMesh(core_axis_name="c", subcore_axis_name="s")

    @functools.partial(
        pl.kernel,
        mesh=mesh,
        compiler_params=pltpu.CompilerParams(
            needs_layout_passes=False, use_tc_tiling_on_sc=True),
        out_type=jax.ShapeDtypeStruct((NB * K * DPATCH,), jnp.float32),
        scratch_types=[
            pltpu.VMEM((ROWS48, W), jnp.float32),
            pltpu.VMEM((ROWS48, W), jnp.float32),
            pltpu.VMEM((RING * CHB,), jnp.float32),
            pltpu.VMEM((Kp,), jnp.int32),
            pltpu.VMEM((ROWS48,), jnp.int32),
            pltpu.VMEM((DPATCH,), jnp.int32),
            [pltpu.SemaphoreType.DMA for _ in range(4)],
        ],
    )
    def sc_kernel(img2d, qcol, starts, iperm, out, strip_a, strip_b, ring,
                  qcol_v, starts_v, iperm_v, sems):
        b = lax.axis_index("s") * 2 + lax.axis_index("c")
        sem_sa, sem_sb, sem_wb, sem_rem = sems
        pltpu.sync_copy(qcol, qcol_v)
        pltpu.sync_copy(starts, starts_v)
        pltpu.sync_copy(iperm, iperm_v)

        def fire_strip(r, strip, sem):
            return [
                pltpu.async_copy(
                    img2d.at[pl.ds((b * CCH + c) * H + PSZ * r, PSZ), :],
                    strip.at[pl.ds(c * PSZ, PSZ), :], sem)
                for c in range(CCH)
            ]

        def drain_strip(strip, sem):
            for c in range(CCH):
                pltpu.make_async_copy(
                    img2d.at[pl.ds(0, PSZ), :],
                    strip.at[pl.ds(c * PSZ, PSZ), :], sem).wait()

        def process(r, strip):
            s0 = _vextract(starts_v, r)
            s1 = _vextract(starts_v, r + 1)

            def pbody(k, carry):
                cb = _vextract(qcol_v, k)
                km = k - (k // (RING * CH)) * (RING * CH)
                dst = ring.at[pl.ds(km * DPATCH, DPATCH)]
                for j in range(ROWS48):
                    ipj = iperm_v[pl.ds(j * PSZ, PSZ)]
                    vec = strip[j, pl.ds(cb, PSZ)]
                    plsc.store_scatter(dst, [ipj], vec)
                return carry

            lax.fori_loop(s0, s1, pbody, 0)

            def fbody(blk, carry):
                @pl.when(blk >= RING - 2)
                def _():
                    pltpu.make_async_copy(
                        out.at[pl.ds(0, CHB)], ring.at[pl.ds(0, CHB)],
                        sem_wb).wait()
                slot = blk - (blk // RING) * RING
                pltpu.async_copy(
                    ring.at[pl.ds(slot * CHB, CHB)],
                    out.at[pl.ds((b * K + blk * CH) * DPATCH, CHB)], sem_wb)
                return carry

            lax.fori_loop(s0 // CH, s1 // CH, fbody, 0)

        h0 = fire_strip(0, strip_a, sem_sa)
        del h0

        def srbody(rr, carry):
            r0 = rr * 2
            hb = fire_strip(r0 + 1, strip_b, sem_sb)
            drain_strip(strip_a, sem_sa)
            process(r0, strip_a)

            @pl.when(r0 + 2 < NR)
            def _():
                fire_strip(r0 + 2, strip_a, sem_sa)

            for h in hb:
                h.wait()
            process(r0 + 1, strip_b)
            return carry

        lax.fori_loop(0, NR // 2, srbody, 0)

        if rem:
            slot = nblk - (nblk // RING) * RING
            pltpu.async_copy(
                ring.at[pl.ds(slot * CHB, rem * DPATCH)],
                out.at[pl.ds((b * K + nblk * CH) * DPATCH, rem * DPATCH)],
                sem_rem)
        for _ in range(nblk - max(0, nblk - (RING - 2))):
            pltpu.make_async_copy(
                out.at[pl.ds(0, CHB)], ring.at[pl.ds(0, CHB)], sem_wb).wait()
        if rem:
            pltpu.make_async_copy(
                out.at[pl.ds(0, rem * DPATCH)],
                ring.at[pl.ds(0, rem * DPATCH)], sem_rem).wait()

    return sc_kernel


def kernel(img, patch_indices):
    K = patch_indices.shape[0]
    Kp = ((K + PSZ) // PSZ) * PSZ
    img2d = img.reshape(NB * CCH * H, W)
    q = (patch_indices % WW).astype(jnp.int32)
    qcol = jnp.zeros((Kp,), jnp.int32).at[:K].set(q * PSZ)
    r = (patch_indices // WW).astype(jnp.int32)
    starts = jnp.searchsorted(r, jnp.arange(ROWS48, dtype=jnp.int32),
                              side="left").astype(jnp.int32)
    out_flat = _make_sc_call(K, Kp)(img2d, qcol, starts,
                                    jnp.asarray(_IPERM))
    return out_flat.reshape(NB, K, DPATCH)


# speedup vs baseline: 4.3777x; 1.2430x over previous
import functools

import jax
import jax.numpy as jnp
import numpy as np
from jax import lax
from jax.experimental import pallas as pl
from jax.experimental.pallas import tpu as pltpu
from jax.experimental.pallas import tpu_sc as plsc

H = 512
W = 512
PSZ = 16
CCH = 3
NB = 32
WW = W // PSZ
NR = H // PSZ
ROWS48 = CCH * PSZ
DPATCH = PSZ * PSZ * CCH

CH = 32
RING = 3
CHB = CH * DPATCH

def _vextract(ref, i):
    base = (i >> 4) << 4
    v = ref[pl.ds(base, PSZ)]
    lane = i - base
    sel = jnp.where(lax.iota(jnp.int32, PSZ) == lane, v, 0)
    return jnp.sum(sel)


@functools.lru_cache(maxsize=None)
def _make_sc_call(K: int, Kp: int):
    nblk, rem = divmod(K, CH)
    mesh = plsc.VectorSubcoreMesh(core_axis_name="c", subcore_axis_name="s")

    @functools.partial(
        pl.kernel,
        mesh=mesh,
        compiler_params=pltpu.CompilerParams(
            needs_layout_passes=False, use_tc_tiling_on_sc=True),
        out_type=jax.ShapeDtypeStruct((NB * K * DPATCH,), jnp.float32),
        scratch_types=[
            pltpu.VMEM((ROWS48, W), jnp.float32),
            pltpu.VMEM((ROWS48, W), jnp.float32),
            pltpu.VMEM((RING * CHB,), jnp.float32),
            pltpu.VMEM((Kp,), jnp.int32),
            pltpu.VMEM((ROWS48,), jnp.int32),
            [pltpu.SemaphoreType.DMA for _ in range(4)],
        ],
    )
    def sc_kernel(img2d, qcol, starts, out, strip_a, strip_b, ring,
                  qcol_v, starts_v, sems):
        b = lax.axis_index("s") * 2 + lax.axis_index("c")
        sem_sa, sem_sb, sem_wb, sem_rem = sems
        pltpu.sync_copy(qcol, qcol_v)
        pltpu.sync_copy(starts, starts_v)

        def fire_strip(r, strip, sem):
            return [
                pltpu.async_copy(
                    img2d.at[pl.ds((b * CCH + c) * H + PSZ * r, PSZ), :],
                    strip.at[pl.ds(c * PSZ, PSZ), :], sem)
                for c in range(CCH)
            ]

        def drain_strip(strip, sem):
            for c in range(CCH):
                pltpu.make_async_copy(
                    img2d.at[pl.ds(0, PSZ), :],
                    strip.at[pl.ds(c * PSZ, PSZ), :], sem).wait()

        def process(r, strip):
            s0 = _vextract(starts_v, r)
            s1 = _vextract(starts_v, r + 1)

            def pbody(k, carry):
                cb = _vextract(qcol_v, k)
                km = k - (k // (RING * CH)) * (RING * CH)
                dst = ring.at[pl.ds(km * DPATCH, DPATCH)]
                iota3 = lax.iota(jnp.int32, PSZ) * CCH
                for j in range(ROWS48):
                    ipj = iota3 + ((j % PSZ) * ROWS48 + j // PSZ)
                    vec = strip[j, pl.ds(cb, PSZ)]
                    plsc.store_scatter(dst, [ipj], vec)
                return carry

            lax.fori_loop(s0, s1, pbody, 0)

            def fbody(blk, carry):
                @pl.when(blk >= RING - 2)
                def _():
                    pltpu.make_async_copy(
                        out.at[pl.ds(0, CHB)], ring.at[pl.ds(0, CHB)],
                        sem_wb).wait()
                slot = blk - (blk // RING) * RING
                pltpu.async_copy(
                    ring.at[pl.ds(slot * CHB, CHB)],
                    out.at[pl.ds((b * K + blk * CH) * DPATCH, CHB)], sem_wb)
                return carry

            lax.fori_loop(s0 // CH, s1 // CH, fbody, 0)

        h0 = fire_strip(0, strip_a, sem_sa)
        del h0

        def srbody(rr, carry):
            r0 = rr * 2
            hb = fire_strip(r0 + 1, strip_b, sem_sb)
            drain_strip(strip_a, sem_sa)
            process(r0, strip_a)

            @pl.when(r0 + 2 < NR)
            def _():
                fire_strip(r0 + 2, strip_a, sem_sa)

            for h in hb:
                h.wait()
            process(r0 + 1, strip_b)
            return carry

        lax.fori_loop(0, NR // 2, srbody, 0)

        if rem:
            slot = nblk - (nblk // RING) * RING
            pltpu.async_copy(
                ring.at[pl.ds(slot * CHB, rem * DPATCH)],
                out.at[pl.ds((b * K + nblk * CH) * DPATCH, rem * DPATCH)],
                sem_rem)
        for _ in range(nblk - max(0, nblk - (RING - 2))):
            pltpu.make_async_copy(
                out.at[pl.ds(0, CHB)], ring.at[pl.ds(0, CHB)], sem_wb).wait()
        if rem:
            pltpu.make_async_copy(
                out.at[pl.ds(0, rem * DPATCH)],
                ring.at[pl.ds(0, rem * DPATCH)], sem_rem).wait()

    return sc_kernel


def kernel(img, patch_indices):
    K = patch_indices.shape[0]
    Kp = ((K + PSZ) // PSZ) * PSZ
    img2d = img.reshape(NB * CCH * H, W)
    q = (patch_indices % WW).astype(jnp.int32)
    qcol = jnp.zeros((Kp,), jnp.int32).at[:K].set(q * PSZ)
    r = (patch_indices // WW).astype(jnp.int32)
    starts = jnp.searchsorted(r, jnp.arange(ROWS48, dtype=jnp.int32),
                              side="left").astype(jnp.int32)
    out_flat = _make_sc_call(K, Kp)(img2d, qcol, starts)
    return out_flat.reshape(NB, K, DPATCH)


# speedup vs baseline: 5.6433x; 1.2891x over previous
import functools

import jax
import jax.numpy as jnp
import numpy as np
from jax import lax
from jax.experimental import pallas as pl
from jax.experimental.pallas import tpu as pltpu
from jax.experimental.pallas import tpu_sc as plsc

H = 512
W = 512
PSZ = 16
CCH = 3
NB = 32
WW = W // PSZ
NR = H // PSZ
ROWS48 = CCH * PSZ
DPATCH = PSZ * PSZ * CCH

CH = 32
RING = 3
CHB = CH * DPATCH

def _vextract(ref, i):
    base = (i >> 4) << 4
    v = ref[pl.ds(base, PSZ)]
    lane = i - base
    sel = jnp.where(lax.iota(jnp.int32, PSZ) == lane, v, 0)
    return jnp.sum(sel)


@functools.lru_cache(maxsize=None)
def _make_sc_call(K: int, Kp: int):
    nblk, rem = divmod(K, CH)
    mesh = plsc.VectorSubcoreMesh(core_axis_name="c", subcore_axis_name="s")

    @functools.partial(
        pl.kernel,
        mesh=mesh,
        compiler_params=pltpu.CompilerParams(
            needs_layout_passes=False, use_tc_tiling_on_sc=True),
        out_type=jax.ShapeDtypeStruct((NB * K * DPATCH,), jnp.float32),
        scratch_types=[
            pltpu.VMEM((ROWS48, W), jnp.float32),
            pltpu.VMEM((ROWS48, W), jnp.float32),
            pltpu.VMEM((RING * CHB,), jnp.float32),
            pltpu.VMEM((Kp,), jnp.int32),
            pltpu.VMEM((ROWS48,), jnp.int32),
            [pltpu.SemaphoreType.DMA for _ in range(4)],
        ],
    )
    def sc_kernel(img2d, qcol, starts, out, strip_a, strip_b, ring,
                  qcol_v, starts_v, sems):
        b = lax.axis_index("s") * 2 + lax.axis_index("c")
        sem_sa, sem_sb, sem_wb, sem_rem = sems
        pltpu.sync_copy(qcol, qcol_v)
        pltpu.sync_copy(starts, starts_v)

        def fire_strip(r, strip, sem):
            return [
                pltpu.async_copy(
                    img2d.at[pl.ds((b * CCH + c) * H + PSZ * r, PSZ), :],
                    strip.at[pl.ds(c * PSZ, PSZ), :], sem)
                for c in range(CCH)
            ]

        def drain_strip(strip, sem):
            for c in range(CCH):
                pltpu.make_async_copy(
                    img2d.at[pl.ds(0, PSZ), :],
                    strip.at[pl.ds(c * PSZ, PSZ), :], sem).wait()

        def process(r, strip):
            s0 = _vextract(starts_v, r)
            s1 = _vextract(starts_v, r + 1)

            @plsc.parallel_loop(s0, s1)
            def pbody(k):
                cb = _vextract(qcol_v, k)
                km = k - (k // (RING * CH)) * (RING * CH)
                dst = ring.at[pl.ds(km * DPATCH, DPATCH)]
                iota3 = lax.iota(jnp.int32, PSZ) * CCH
                for j in range(ROWS48):
                    ipj = iota3 + ((j % PSZ) * ROWS48 + j // PSZ)
                    vec = strip[j, pl.ds(cb, PSZ)]
                    plsc.store_scatter(dst, [ipj], vec)

            def fbody(blk, carry):
                @pl.when(blk >= RING - 2)
                def _():
                    pltpu.make_async_copy(
                        out.at[pl.ds(0, CHB)], ring.at[pl.ds(0, CHB)],
                        sem_wb).wait()
                slot = blk - (blk // RING) * RING
                pltpu.async_copy(
                    ring.at[pl.ds(slot * CHB, CHB)],
                    out.at[pl.ds((b * K + blk * CH) * DPATCH, CHB)], sem_wb)
                return carry

            lax.fori_loop(s0 // CH, s1 // CH, fbody, 0)

        h0 = fire_strip(0, strip_a, sem_sa)
        del h0

        def srbody(rr, carry):
            r0 = rr * 2
            hb = fire_strip(r0 + 1, strip_b, sem_sb)
            drain_strip(strip_a, sem_sa)
            process(r0, strip_a)

            @pl.when(r0 + 2 < NR)
            def _():
                fire_strip(r0 + 2, strip_a, sem_sa)

            for h in hb:
                h.wait()
            process(r0 + 1, strip_b)
            return carry

        lax.fori_loop(0, NR // 2, srbody, 0)

        if rem:
            slot = nblk - (nblk // RING) * RING
            pltpu.async_copy(
                ring.at[pl.ds(slot * CHB, rem * DPATCH)],
                out.at[pl.ds((b * K + nblk * CH) * DPATCH, rem * DPATCH)],
                sem_rem)
        for _ in range(nblk - max(0, nblk - (RING - 2))):
            pltpu.make_async_copy(
                out.at[pl.ds(0, CHB)], ring.at[pl.ds(0, CHB)], sem_wb).wait()
        if rem:
            pltpu.make_async_copy(
                out.at[pl.ds(0, rem * DPATCH)],
                ring.at[pl.ds(0, rem * DPATCH)], sem_rem).wait()

    return sc_kernel


def kernel(img, patch_indices):
    K = patch_indices.shape[0]
    Kp = ((K + PSZ) // PSZ) * PSZ
    img2d = img.reshape(NB * CCH * H, W)
    q = (patch_indices % WW).astype(jnp.int32)
    qcol = jnp.zeros((Kp,), jnp.int32).at[:K].set(q * PSZ)
    r = (patch_indices // WW).astype(jnp.int32)
    starts = jnp.searchsorted(r, jnp.arange(ROWS48, dtype=jnp.int32),
                              side="left").astype(jnp.int32)
    out_flat = _make_sc_call(K, Kp)(img2d, qcol, starts)
    return out_flat.reshape(NB, K, DPATCH)


# speedup vs baseline: 7.8989x; 1.3997x over previous
import functools

import jax
import jax.numpy as jnp
import numpy as np
from jax import lax
from jax.experimental import pallas as pl
from jax.experimental.pallas import tpu as pltpu
from jax.experimental.pallas import tpu_sc as plsc

H = 512
W = 512
PSZ = 16
CCH = 3
NB = 32
WW = W // PSZ
NR = H // PSZ
ROWS48 = CCH * PSZ
DPATCH = PSZ * PSZ * CCH

CH = 32
RING = 3
CHB = CH * DPATCH

def _vextract(ref, i):
    base = (i >> 4) << 4
    v = ref[pl.ds(base, PSZ)]
    lane = i - base
    sel = jnp.where(lax.iota(jnp.int32, PSZ) == lane, v, 0)
    return jnp.sum(sel)


@functools.lru_cache(maxsize=None)
def _make_sc_call(K: int, Kp: int):
    nblk, rem = divmod(K, CH)
    mesh = plsc.VectorSubcoreMesh(core_axis_name="c", subcore_axis_name="s")

    @functools.partial(
        pl.kernel,
        mesh=mesh,
        compiler_params=pltpu.CompilerParams(
            needs_layout_passes=False, use_tc_tiling_on_sc=True),
        out_type=jax.ShapeDtypeStruct((NB, K, DPATCH), jnp.float32),
        scratch_types=[
            pltpu.VMEM((ROWS48, W), jnp.float32),
            pltpu.VMEM((ROWS48, W), jnp.float32),
            pltpu.VMEM((RING * CH, DPATCH), jnp.float32),
            pltpu.VMEM((Kp,), jnp.int32),
            pltpu.VMEM((ROWS48,), jnp.int32),
            [pltpu.SemaphoreType.DMA for _ in range(4)],
        ],
    )
    def sc_kernel(img2d, qcol, starts, out, strip_a, strip_b, ring,
                  qcol_v, starts_v, sems):
        b = lax.axis_index("s") * 2 + lax.axis_index("c")
        sem_sa, sem_sb, sem_wb, sem_rem = sems
        pltpu.sync_copy(qcol, qcol_v)
        pltpu.sync_copy(starts, starts_v)

        def fire_strip(r, strip, sem):
            return [
                pltpu.async_copy(
                    img2d.at[pl.ds((b * CCH + c) * H + PSZ * r, PSZ), :],
                    strip.at[pl.ds(c * PSZ, PSZ), :], sem)
                for c in range(CCH)
            ]

        def drain_strip(strip, sem):
            for c in range(CCH):
                pltpu.make_async_copy(
                    img2d.at[pl.ds(0, PSZ), :],
                    strip.at[pl.ds(c * PSZ, PSZ), :], sem).wait()

        def process(r, strip):
            s0 = _vextract(starts_v, r)
            s1 = _vextract(starts_v, r + 1)

            @plsc.parallel_loop(s0, s1)
            def pbody(k):
                cb = _vextract(qcol_v, k)
                km = k - (k // (RING * CH)) * (RING * CH)
                rowv = jnp.full((PSZ,), km, jnp.int32)
                iota3 = lax.iota(jnp.int32, PSZ) * CCH
                for j in range(ROWS48):
                    ipj = iota3 + ((j % PSZ) * ROWS48 + j // PSZ)
                    vec = strip[j, pl.ds(cb, PSZ)]
                    plsc.store_scatter(ring, [rowv, ipj], vec)

            def fbody(blk, carry):
                @pl.when(blk >= RING - 2)
                def _():
                    pltpu.make_async_copy(
                        out.at[0, pl.ds(0, CH), :], ring.at[pl.ds(0, CH), :],
                        sem_wb).wait()
                slot = blk - (blk // RING) * RING
                pltpu.async_copy(
                    ring.at[pl.ds(slot * CH, CH), :],
                    out.at[b, pl.ds(blk * CH, CH), :], sem_wb)
                return carry

            lax.fori_loop(s0 // CH, s1 // CH, fbody, 0)

        h0 = fire_strip(0, strip_a, sem_sa)
        del h0

        def srbody(rr, carry):
            r0 = rr * 2
            hb = fire_strip(r0 + 1, strip_b, sem_sb)
            drain_strip(strip_a, sem_sa)
            process(r0, strip_a)

            @pl.when(r0 + 2 < NR)
            def _():
                fire_strip(r0 + 2, strip_a, sem_sa)

            for h in hb:
                h.wait()
            process(r0 + 1, strip_b)
            return carry

        lax.fori_loop(0, NR // 2, srbody, 0)

        if rem:
            slot = nblk - (nblk // RING) * RING
            pltpu.async_copy(
                ring.at[pl.ds(slot * CH, rem), :],
                out.at[b, pl.ds(nblk * CH, rem), :], sem_rem)
        for _ in range(nblk - max(0, nblk - (RING - 2))):
            pltpu.make_async_copy(
                out.at[0, pl.ds(0, CH), :], ring.at[pl.ds(0, CH), :],
                sem_wb).wait()
        if rem:
            pltpu.make_async_copy(
                out.at[0, pl.ds(0, rem), :], ring.at[pl.ds(0, rem), :],
                sem_rem).wait()

    return sc_kernel


def kernel(img, patch_indices):
    K = patch_indices.shape[0]
    Kp = ((K + PSZ) // PSZ) * PSZ
    img2d = img.reshape(NB * CCH * H, W)
    q = (patch_indices % WW).astype(jnp.int32)
    qcol = jnp.zeros((Kp,), jnp.int32).at[:K].set(q * PSZ)
    r = (patch_indices // WW).astype(jnp.int32)
    starts = jnp.sum(r[None, :] < jnp.arange(ROWS48, dtype=jnp.int32)[:, None],
                     axis=1, dtype=jnp.int32)
    return _make_sc_call(K, Kp)(img2d, qcol, starts)
